# baseline (speedup 1.0000x reference)
import functools

import jax
import jax.numpy as jnp
from jax import lax
from jax.experimental import pallas as pl
from jax.experimental.pallas import tpu as pltpu
from jax.experimental.pallas import tpu_sc as plsc

N = 16384
D = 64
V = 1000000
NUM_CORES = 2
NUM_SUBCORES = 16
NW = NUM_CORES * NUM_SUBCORES
B_PER_W = N // NW
L = 16

NQ = (V + 127) // 128
QPW = (NQ + NW - 1) // NW
T = 4
NCH = (QPW + T - 1) // T
QLAST = NQ - 1
QCMAX = (V - 128 * T) // 128
VLAST = QLAST * 128

NG = N // L
CAP = N + L
JMASK = N - 1


def _kernel_a(fnums, tbl_t):
    mesh = plsc.VectorSubcoreMesh(core_axis_name="c", subcore_axis_name="s")

    @functools.partial(
        pl.kernel,
        mesh=mesh,
        out_type=jax.ShapeDtypeStruct((N * D,), jnp.float32),
        scratch_types=[
            pltpu.VMEM((N,), jnp.int32),
            pltpu.VMEM((CAP,), jnp.int32),
            pltpu.VMEM((CAP,), jnp.int32),
            pltpu.VMEM((2, D, 128 * T), jnp.float32),
            pltpu.VMEM((L, D), jnp.float32),
            pltpu.VMEM((D, V - VLAST), jnp.float32),
            pltpu.VMEM((D,), jnp.float32),
            pltpu.SemaphoreType.DMA,
            pltpu.SemaphoreType.DMA,
        ],
        compiler_params=pltpu.CompilerParams(needs_layout_passes=False),
    )
    def ka(fnums_hbm, tbl_hbm, tail_hbm, x_hbm, all_v, ml_v, cl_v, ring_v,
           col_v, tail_v, drain_v, semf, semc):
        wid = lax.axis_index("s") * NUM_CORES + lax.axis_index("c")
        lo = wid * QPW
        hi = jnp.minimum(lo + QPW, NQ)
        iota = lax.iota(jnp.int32, L)
        zeros = jnp.zeros((L,), jnp.int32)

        lo128 = lo * 128
        hi128 = hi * 128

        def filt(g, mcnt):
            ivec = all_v[pl.ds(g * L, L)]
            jv = iota + g * L
            m = (ivec >= lo128) & (ivec < hi128)
            w = lax.shift_left(ivec - lo128, 14) | jv
            plsc.store_compressed(ml_v.at[pl.ds(mcnt, L)], w, mask=m)
            return mcnt + plsc.all_reduce_population_count(m)[0]

        def chunk_base(c):
            return jnp.minimum(lo + c * T, QCMAX)

        def fetch(c, sl):
            lane = pl.multiple_of(chunk_base(c) * 128, 128)
            for s in range(8):
                pltpu.async_copy(
                    tbl_hbm.at[pl.ds(s * 8, 8), pl.ds(lane, 128 * T)],
                    ring_v.at[sl].at[pl.ds(s * 8, 8), :],
                    semf,
                )

        fcopy = pltpu.async_copy(fnums_hbm, all_v, semc)
        fetch(0, 0)
        fcopy.wait()
        mcnt = lax.fori_loop(0, NG, filt, jnp.int32(0))
        mgrp = (mcnt + (L - 1)) // L


        def extract_batches(ccnt, gather_fn):
            nb = (ccnt + (L - 1)) // L

            def batch(b, _):
                nfire = jnp.minimum(ccnt - b * L, L)
                for u in range(L):
                    p = b * L + u

                    @pl.when(p < ccnt)
                    def _():
                        w = cl_v[pl.ds(p, L)][0]
                        off = lax.shift_right_logical(w, 14)
                        j = w & JMASK
                        cv = col_v.at[u]
                        for kk in range(D // L):
                            cv[pl.ds(kk * L, L)] = gather_fn(kk, off)
                        pltpu.async_copy(cv, x_hbm.at[pl.ds(j * D, D)], semc)

                def dr(_, __):
                    pltpu.make_async_copy(x_hbm.at[pl.ds(0, D)], drain_v,
                                          semc).wait()
                    return 0

                lax.fori_loop(0, nfire, dr, 0)
                return 0

            lax.fori_loop(0, nb, batch, 0)

        def per_chunk(c, _):
            sl = c & 1
            lane = pl.multiple_of(chunk_base(c) * 128, 128)
            for s in range(8):
                pltpu.make_async_copy(
                    tbl_hbm.at[pl.ds(s * 8, 8), pl.ds(lane, 128 * T)],
                    ring_v.at[sl].at[pl.ds(s * 8, 8), :], semf).wait()

            @pl.when(c + 1 < NCH)
            def _():
                fetch(c + 1, 1 - sl)

            nom = lo + c * T
            qcb = chunk_base(c)

            def sel(g, ccnt):
                w = ml_v[pl.ds(g * L, L)]
                q = lax.shift_right_logical(w, 21) + lo
                m = ((q >= nom) & (q < nom + T) & (q >= qcb) & (q < qcb + T)
                     & ((iota + g * L) < mcnt))
                r = lax.shift_right_logical(w, 14) & 127
                w2 = lax.shift_left((q - qcb) * 128 + r, 14) | (w & JMASK)
                plsc.store_compressed(cl_v.at[pl.ds(ccnt, L)], w2, mask=m)
                return ccnt + plsc.all_reduce_population_count(m)[0]

            ccnt = lax.fori_loop(0, mgrp, sel, jnp.int32(0))

            def gather_fn(kk, off):
                return plsc.load_gather(
                    ring_v, [zeros + sl, iota + kk * L, zeros + off])

            extract_batches(ccnt, gather_fn)
            return 0

        lax.fori_loop(0, NCH, per_chunk, 0)

        @pl.when((lo <= QLAST) & (QLAST < hi))
        def _():
            pltpu.sync_copy(tail_hbm, tail_v)

            def sel(g, ccnt):
                w = ml_v[pl.ds(g * L, L)]
                q = lax.shift_right_logical(w, 21) + lo
                m = (q == QLAST) & ((iota + g * L) < mcnt)
                r = lax.shift_right_logical(w, 14) & 127
                w2 = lax.shift_left(r, 14) | (w & JMASK)
                plsc.store_compressed(cl_v.at[pl.ds(ccnt, L)], w2, mask=m)
                return ccnt + plsc.all_reduce_population_count(m)[0]

            ccnt = lax.fori_loop(0, mgrp, sel, jnp.int32(0))

            def gather_fn(kk, off):
                return plsc.load_gather(tail_v, [iota + kk * L, zeros + off])

            extract_batches(ccnt, gather_fn)

    tail = lax.slice(tbl_t, (0, VLAST), (D, V))
    return ka(fnums, tbl_t, tail)


def _kernel_b(x):
    mesh = plsc.VectorSubcoreMesh(core_axis_name="c", subcore_axis_name="s")

    @functools.partial(
        pl.kernel,
        mesh=mesh,
        out_type=jax.ShapeDtypeStruct((D, N), jnp.float32),
        scratch_types=[
            pltpu.VMEM((B_PER_W * D,), jnp.float32),
            pltpu.VMEM((D, B_PER_W), jnp.float32),
        ],
        compiler_params=pltpu.CompilerParams(needs_layout_passes=False),
    )
    def kb(x_hbm, out_hbm, rows_v, cols_v):
        wid = lax.axis_index("s") * NUM_CORES + lax.axis_index("c")
        base = wid * B_PER_W
        pltpu.sync_copy(x_hbm.at[pl.ds(base * D, B_PER_W * D)], rows_v)
        riota = lax.iota(jnp.int32, L)

        def body(j, _):
            jvec = jnp.zeros((L,), dtype=jnp.int32) + j
            for kk in range(D // L):
                vals = rows_v[pl.ds(j * D + kk * L, L)]
                plsc.store_scatter(cols_v, [riota + kk * L, jvec], vals)
            return 0

        lax.fori_loop(0, B_PER_W, body, 0)
        pltpu.sync_copy(cols_v, out_hbm.at[:, pl.ds(base, B_PER_W)])

    return kb(x)


def kernel(fnums, emb_table):
    x = _kernel_a(fnums.astype(jnp.int32), emb_table.T)
    return x.reshape(N, D)


# speedup vs baseline: 1.0055x; 1.0055x over previous
import functools

import jax
import jax.numpy as jnp
from jax import lax
from jax.experimental import pallas as pl
from jax.experimental.pallas import tpu as pltpu
from jax.experimental.pallas import tpu_sc as plsc

N = 16384
D = 64
V = 1000000
NUM_CORES = 2
NUM_SUBCORES = 16
NW = NUM_CORES * NUM_SUBCORES
L = 16

NQ = (V + 127) // 128
QPW = (NQ + NW - 1) // NW
T = 4
NCH = (QPW + T - 1) // T
QLAST = NQ - 1
QCMAX = (V - 128 * T) // 128
VLAST = QLAST * 128

NG = N // L
CAP = N + L
JMASK = N - 1


def _kernel_a(fnums, tbl_t):
    mesh = plsc.VectorSubcoreMesh(core_axis_name="c", subcore_axis_name="s")

    @functools.partial(
        pl.kernel,
        mesh=mesh,
        out_type=jax.ShapeDtypeStruct((N * D,), jnp.float32),
        scratch_types=[
            pltpu.VMEM((N,), jnp.int32),
            pltpu.VMEM((CAP,), jnp.int32),
            pltpu.VMEM((CAP,), jnp.int32),
            pltpu.VMEM((2, D, 128 * T), jnp.float32),
            pltpu.VMEM((L, D), jnp.float32),
            pltpu.VMEM((D, V - VLAST), jnp.float32),
            pltpu.VMEM((D,), jnp.float32),
            pltpu.SemaphoreType.DMA,
            pltpu.SemaphoreType.DMA,
        ],
        compiler_params=pltpu.CompilerParams(needs_layout_passes=False),
    )
    def ka(fnums_hbm, tbl_hbm, tail_hbm, x_hbm, all_v, ml_v, cl_v, ring_v,
           col_v, tail_v, drain_v, semf, semc):
        wid = lax.axis_index("s") * NUM_CORES + lax.axis_index("c")
        lo = wid * QPW
        hi = jnp.minimum(lo + QPW, NQ)
        iota = lax.iota(jnp.int32, L)
        zeros = jnp.zeros((L,), jnp.int32)

        lo128 = lo * 128
        hi128 = hi * 128

        def filt(g, mcnt):
            ivec = all_v[pl.ds(g * L, L)]
            jv = iota + g * L
            m = (ivec >= lo128) & (ivec < hi128)
            w = lax.shift_left(ivec - lo128, 14) | jv
            plsc.store_compressed(ml_v.at[pl.ds(mcnt, L)], w, mask=m)
            return mcnt + plsc.all_reduce_population_count(m)[0]

        def chunk_base(c):
            return jnp.minimum(lo + c * T, QCMAX)

        def fetch(c, sl):
            lane = pl.multiple_of(chunk_base(c) * 128, 128)
            for s in range(8):
                pltpu.async_copy(
                    tbl_hbm.at[pl.ds(s * 8, 8), pl.ds(lane, 128 * T)],
                    ring_v.at[sl].at[pl.ds(s * 8, 8), :],
                    semf,
                )

        fcopy = pltpu.async_copy(fnums_hbm, all_v, semc)
        fetch(0, 0)
        fcopy.wait()
        mcnt = lax.fori_loop(0, NG, filt, jnp.int32(0))
        mgrp = (mcnt + (L - 1)) // L


        def extract_batches(ccnt, gather_fn):
            nb = (ccnt + (L - 1)) // L

            def batch(b, _):
                nfire = jnp.minimum(ccnt - b * L, L)
                for u in range(L):
                    p = b * L + u

                    @pl.when(p < ccnt)
                    def _():
                        w = cl_v[pl.ds(p, L)][0]
                        off = lax.shift_right_logical(w, 14)
                        j = w & JMASK
                        cv = col_v.at[u]
                        for kk in range(D // L):
                            cv[pl.ds(kk * L, L)] = gather_fn(kk, off)
                        pltpu.async_copy(cv, x_hbm.at[pl.ds(j * D, D)], semc)

                def dr(_, __):
                    pltpu.make_async_copy(x_hbm.at[pl.ds(0, D)], drain_v,
                                          semc).wait()
                    return 0

                lax.fori_loop(0, nfire, dr, 0)
                return 0

            lax.fori_loop(0, nb, batch, 0)

        def per_chunk(c, _):
            sl = c & 1
            lane = pl.multiple_of(chunk_base(c) * 128, 128)
            for s in range(8):
                pltpu.make_async_copy(
                    tbl_hbm.at[pl.ds(s * 8, 8), pl.ds(lane, 128 * T)],
                    ring_v.at[sl].at[pl.ds(s * 8, 8), :], semf).wait()

            @pl.when(c + 1 < NCH)
            def _():
                fetch(c + 1, 1 - sl)

            nom = lo + c * T
            qcb = chunk_base(c)

            def sel(g, ccnt):
                w = ml_v[pl.ds(g * L, L)]
                q = lax.shift_right_logical(w, 21) + lo
                m = ((q >= nom) & (q < nom + T) & (q >= qcb) & (q < qcb + T)
                     & ((iota + g * L) < mcnt))
                r = lax.shift_right_logical(w, 14) & 127
                w2 = lax.shift_left((q - qcb) * 128 + r, 14) | (w & JMASK)
                plsc.store_compressed(cl_v.at[pl.ds(ccnt, L)], w2, mask=m)
                return ccnt + plsc.all_reduce_population_count(m)[0]

            ccnt = lax.fori_loop(0, mgrp, sel, jnp.int32(0))

            def gather_fn(kk, off):
                return plsc.load_gather(
                    ring_v, [zeros + sl, iota + kk * L, zeros + off])

            extract_batches(ccnt, gather_fn)
            return 0

        lax.fori_loop(0, NCH, per_chunk, 0)

        @pl.when((lo <= QLAST) & (QLAST < hi))
        def _():
            pltpu.sync_copy(tail_hbm, tail_v)

            def sel(g, ccnt):
                w = ml_v[pl.ds(g * L, L)]
                q = lax.shift_right_logical(w, 21) + lo
                m = (q == QLAST) & ((iota + g * L) < mcnt)
                r = lax.shift_right_logical(w, 14) & 127
                w2 = lax.shift_left(r, 14) | (w & JMASK)
                plsc.store_compressed(cl_v.at[pl.ds(ccnt, L)], w2, mask=m)
                return ccnt + plsc.all_reduce_population_count(m)[0]

            ccnt = lax.fori_loop(0, mgrp, sel, jnp.int32(0))

            def gather_fn(kk, off):
                return plsc.load_gather(tail_v, [iota + kk * L, zeros + off])

            extract_batches(ccnt, gather_fn)

    tail = lax.slice(tbl_t, (0, VLAST), (D, V))
    return ka(fnums, tbl_t, tail)


def kernel(fnums, emb_table):
    x = _kernel_a(fnums.astype(jnp.int32), emb_table.T)
    return x.reshape(N, D)
